# Initial kernel scaffold; baseline (speedup 1.0000x reference)
import jax
import jax.numpy as jnp
from jax.experimental import pallas as pl


def kernel(x, table):
    raise NotImplementedError("write your pallas kernel here")


# speedup vs baseline: 2.6208x; 2.6208x over previous
import functools

import jax
import jax.numpy as jnp
from jax import lax
from jax.experimental import pallas as pl
from jax.experimental.pallas import tpu as pltpu
from jax.experimental.pallas import tpu_sc as plsc

NUM_CODONS = 64
EMBED_DIM = 128

_INFO = plsc.get_sparse_core_info()
_NC = _INFO.num_cores
_NS = _INFO.num_subcores
_NW = _NC * _NS

_CHUNK = 128


def _sc_gather(n_total: int):
  b_per_w = n_total // _NW
  n_chunks = b_per_w // _CHUNK
  mesh = plsc.VectorSubcoreMesh(core_axis_name="c", subcore_axis_name="s")

  @functools.partial(
      pl.kernel,
      mesh=mesh,
      out_type=jax.ShapeDtypeStruct((n_total, EMBED_DIM), jnp.float32),
      scratch_types=[
          pltpu.VMEM((_CHUNK,), jnp.int32),
          pltpu.VMEM((_CHUNK, EMBED_DIM), jnp.float32),
          pltpu.SemaphoreType.DMA,
      ],
  )
  def k(idx_hbm, table_hbm, out_hbm, idx_v, rows_v, sem):
    wid = lax.axis_index("s") * _NC + lax.axis_index("c")
    w_base = wid * b_per_w

    def body(g, _):
      base = w_base + g * _CHUNK
      pltpu.sync_copy(idx_hbm.at[pl.ds(base, _CHUNK)], idx_v)
      pltpu.async_copy(table_hbm.at[idx_v], rows_v, sem).wait()
      pltpu.sync_copy(rows_v, out_hbm.at[pl.ds(base, _CHUNK)])
      return 0

    lax.fori_loop(0, n_chunks, body, 0)

  return k


def kernel(x, table):
  batch, seqlen = x.shape
  n_total = batch * seqlen
  idx = x.reshape((n_total,))
  out = _sc_gather(n_total)(idx, table)
  return out.reshape((batch, seqlen, EMBED_DIM))


# speedup vs baseline: 2.6761x; 1.0211x over previous
import functools

import jax
import jax.numpy as jnp
from jax import lax
from jax.experimental import pallas as pl
from jax.experimental.pallas import tpu as pltpu
from jax.experimental.pallas import tpu_sc as plsc

NUM_CODONS = 64
EMBED_DIM = 128

_INFO = plsc.get_sparse_core_info()
_NC = _INFO.num_cores
_NS = _INFO.num_subcores
_NW = _NC * _NS

_CHUNK = 128
_NBUF = 4
_PAIR = 16


def _sc_gather(n_total: int):
  n_chunks = n_total // _CHUNK
  c_per_w = n_chunks // _NW
  n_g4 = c_per_w // _PAIR
  mesh = plsc.VectorSubcoreMesh(core_axis_name="c", subcore_axis_name="s")

  @functools.partial(
      pl.kernel,
      mesh=mesh,
      out_type=jax.ShapeDtypeStruct((n_total, EMBED_DIM), jnp.float32),
      scratch_types=(
          [pltpu.VMEM((_CHUNK, EMBED_DIM), jnp.float32)] * _NBUF
          + [pltpu.VMEM((8, _CHUNK), jnp.int32)] * 2
          + [pltpu.SemaphoreType.DMA] * (_NBUF + _NBUF + 2)
      ),
  )
  def k(idx_hbm, table_hbm, out_hbm, *refs):
    rows = refs[:_NBUF]
    ibuf = refs[_NBUF:_NBUF + 2]
    sem_g = refs[_NBUF + 2:2 * _NBUF + 2]
    sem_o = refs[2 * _NBUF + 2:3 * _NBUF + 2]
    sem_i = refs[3 * _NBUF + 2:]

    wid = lax.axis_index("s") * _NC + lax.axis_index("c")
    w_chunk0 = wid * c_per_w

    def idx_fetch(pair, buf, sem):
      return pltpu.make_async_copy(
          idx_hbm.at[pl.ds(w_chunk0 + pair * 8, 8)], buf, sem)

    def gather(j, c, hh):
      return pltpu.make_async_copy(
          table_hbm.at[ibuf[hh].at[j]], rows[j % _NBUF], sem_g[j % _NBUF])

    def out_copy(c, slot):
      return pltpu.make_async_copy(
          rows[slot], out_hbm.at[pl.ds(c * _CHUNK, _CHUNK)], sem_o[slot])

    def gather_wait(slot):
      pltpu.make_async_copy(
          table_hbm.at[ibuf[0].at[0]], rows[slot], sem_g[slot]).wait()

    idx_fetch(0, ibuf[0], sem_i[0]).start()

    def body(g4, _):
      for hh in range(2):
        pair = 2 * g4 + hh
        for j in range(8):
          c = pair * 8 + j
          slot = j % _NBUF
          gc = w_chunk0 + c

          if hh == 0 and j < _NBUF:
            @pl.when(g4 > 0)
            def _():
              out_copy(0, slot).wait()
          else:
            out_copy(0, slot).wait()

          if j == 0:
            idx_fetch(0, ibuf[hh], sem_i[hh]).wait()

          gather(j, gc, hh).start()

          pslot = (j - 1) % _NBUF
          if hh == 0 and j == 0:
            @pl.when(g4 > 0)
            def _():
              gather_wait(pslot)
              out_copy(gc - 1, pslot).start()
          else:
            gather_wait(pslot)
            out_copy(gc - 1, pslot).start()

          if j == 0:
            if hh == 0:
              idx_fetch(pair + 1, ibuf[1 - hh], sem_i[1 - hh]).start()
            else:
              @pl.when(g4 < n_g4 - 1)
              def _():
                idx_fetch(pair + 1, ibuf[1 - hh], sem_i[1 - hh]).start()
      return 0

    lax.fori_loop(0, n_g4, body, 0)

    last = w_chunk0 + c_per_w - 1
    gather_wait(3)
    out_copy(last, 3).start()
    for slot in range(_NBUF):
      out_copy(0, slot).wait()

  return k


def kernel(x, table):
  batch, seqlen = x.shape
  n_total = batch * seqlen
  idx = x.reshape((n_total // _CHUNK, _CHUNK))
  out = _sc_gather(n_total)(idx, table)
  return out.reshape((batch, seqlen, EMBED_DIM))


# speedup vs baseline: 19.5408x; 7.3020x over previous
import functools

import jax
import jax.numpy as jnp
from jax import lax
from jax.experimental import pallas as pl
from jax.experimental.pallas import tpu as pltpu
from jax.experimental.pallas import tpu_sc as plsc

NUM_CODONS = 64
EMBED_DIM = 128

_INFO = plsc.get_sparse_core_info()
_NC = _INFO.num_cores
_NS = _INFO.num_subcores
_NW = _NC * _NS

_CHUNK = 128
_NBUF = 4
_PAIR = 16


def _sc_gather(n_total: int):
  n_chunks = n_total // _CHUNK
  c_per_w = n_chunks // _NW
  n_g4 = c_per_w // _PAIR
  mesh = plsc.VectorSubcoreMesh(core_axis_name="c", subcore_axis_name="s")

  @functools.partial(
      pl.kernel,
      mesh=mesh,
      out_type=jax.ShapeDtypeStruct((n_total, EMBED_DIM), jnp.float32),
      scratch_types=(
          [pltpu.VMEM((_CHUNK, EMBED_DIM), jnp.float32)] * _NBUF
          + [pltpu.VMEM((8, _CHUNK), jnp.int32)] * 2
          + [pltpu.VMEM_SHARED((NUM_CODONS, EMBED_DIM), jnp.float32)]
          + [pltpu.SemaphoreType.DMA] * (_NBUF + _NBUF + 2)
      ),
  )
  def k(idx_hbm, table_hbm, out_hbm, *refs):
    rows = refs[:_NBUF]
    ibuf = refs[_NBUF:_NBUF + 2]
    table_v = refs[_NBUF + 2]
    sem_g = refs[_NBUF + 3:2 * _NBUF + 3]
    sem_o = refs[2 * _NBUF + 3:3 * _NBUF + 3]
    sem_i = refs[3 * _NBUF + 3:]

    wid = lax.axis_index("s") * _NC + lax.axis_index("c")
    w_chunk0 = wid * c_per_w

    def idx_fetch(pair, buf, sem):
      return pltpu.make_async_copy(
          idx_hbm.at[pl.ds(w_chunk0 + pair * 8, 8)], buf, sem)

    def gather(j, c, hh):
      return pltpu.make_async_copy(
          table_v.at[ibuf[hh].at[j]], rows[j % _NBUF], sem_g[j % _NBUF])

    def out_copy(c, slot):
      return pltpu.make_async_copy(
          rows[slot], out_hbm.at[pl.ds(c * _CHUNK, _CHUNK)], sem_o[slot])

    def gather_wait(slot):
      pltpu.make_async_copy(
          table_v.at[ibuf[0].at[0]], rows[slot], sem_g[slot]).wait()

    @pl.when(lax.axis_index("s") == 0)
    def _():
      pltpu.sync_copy(table_hbm, table_v)
    plsc.subcore_barrier()

    idx_fetch(0, ibuf[0], sem_i[0]).start()

    def body(g4, _):
      for hh in range(2):
        pair = 2 * g4 + hh
        for j in range(8):
          c = pair * 8 + j
          slot = j % _NBUF
          gc = w_chunk0 + c

          if hh == 0 and j < _NBUF:
            @pl.when(g4 > 0)
            def _():
              out_copy(0, slot).wait()
          else:
            out_copy(0, slot).wait()

          if j == 0:
            idx_fetch(0, ibuf[hh], sem_i[hh]).wait()

          gather(j, gc, hh).start()

          pslot = (j - 1) % _NBUF
          if hh == 0 and j == 0:
            @pl.when(g4 > 0)
            def _():
              gather_wait(pslot)
              out_copy(gc - 1, pslot).start()
          else:
            gather_wait(pslot)
            out_copy(gc - 1, pslot).start()

          if j == 0:
            if hh == 0:
              idx_fetch(pair + 1, ibuf[1 - hh], sem_i[1 - hh]).start()
            else:
              @pl.when(g4 < n_g4 - 1)
              def _():
                idx_fetch(pair + 1, ibuf[1 - hh], sem_i[1 - hh]).start()
      return 0

    lax.fori_loop(0, n_g4, body, 0)

    last = w_chunk0 + c_per_w - 1
    gather_wait(3)
    out_copy(last, 3).start()
    for slot in range(_NBUF):
      out_copy(0, slot).wait()

  return k


def kernel(x, table):
  batch, seqlen = x.shape
  n_total = batch * seqlen
  idx = x.reshape((n_total // _CHUNK, _CHUNK))
  out = _sc_gather(n_total)(idx, table)
  return out.reshape((batch, seqlen, EMBED_DIM))


# speedup vs baseline: 19.6037x; 1.0032x over previous
import functools

import jax
import jax.numpy as jnp
from jax import lax
from jax.experimental import pallas as pl
from jax.experimental.pallas import tpu as pltpu
from jax.experimental.pallas import tpu_sc as plsc

NUM_CODONS = 64
EMBED_DIM = 128

_INFO = plsc.get_sparse_core_info()
_NC = _INFO.num_cores
_NS = _INFO.num_subcores
_NW = _NC * _NS

_CHUNK = 128


def _sc_gather(n_total: int):
  n_chunks = n_total // _CHUNK
  c_per_w = n_chunks // _NW
  n_pairs = c_per_w // 8
  n_g2 = n_pairs // 2
  mesh = plsc.VectorSubcoreMesh(core_axis_name="c", subcore_axis_name="s")

  @functools.partial(
      pl.kernel,
      mesh=mesh,
      out_type=jax.ShapeDtypeStruct((n_chunks, _CHUNK, EMBED_DIM), jnp.float32),
      scratch_types=(
          [pltpu.VMEM((4, _CHUNK, EMBED_DIM), jnp.float32)]
          + [pltpu.VMEM((8, _CHUNK), jnp.int32)] * 2
          + [pltpu.VMEM_SHARED((NUM_CODONS, EMBED_DIM), jnp.float32)]
          + [pltpu.SemaphoreType.DMA] * (4 + 2 + 2)
      ),
  )
  def k(idx_hbm, table_hbm, out_hbm, rows, ibuf0, ibuf1, table_sh, *sems):
    ibuf = (ibuf0, ibuf1)
    sem_g = sems[:4]
    sem_o = sems[4:6]
    sem_i = sems[6:]

    wid = lax.axis_index("s") * _NC + lax.axis_index("c")
    w_chunk0 = wid * c_per_w

    def idx_fetch(pair, buf, sem):
      return pltpu.make_async_copy(
          idx_hbm.at[pl.ds(w_chunk0 + pair * 8, 8)], buf, sem)

    def gather(j, slot, hh):
      return pltpu.make_async_copy(
          table_sh.at[ibuf[hh].at[j]], rows.at[slot], sem_g[slot])

    def gather_wait(slot):
      pltpu.make_async_copy(
          table_sh.at[ibuf[0].at[0]], rows.at[slot], sem_g[slot]).wait()

    def out_copy(blk_gc0, s):
      return pltpu.make_async_copy(
          rows.at[pl.ds(2 * s, 2)], out_hbm.at[pl.ds(blk_gc0, 2)], sem_o[s])

    @pl.when(lax.axis_index("s") == 0)
    def _():
      pltpu.sync_copy(table_hbm, table_sh)
    plsc.subcore_barrier()

    idx_fetch(0, ibuf[0], sem_i[0]).start()

    def body(g2, _):
      for hh in range(2):
        pair = 2 * g2 + hh
        for kk in range(4):
          s = kk % 2
          blk = pair * 4 + kk
          first_ever = hh == 0 and kk < 2

          if first_ever:
            @pl.when(g2 > 0)
            def _():
              out_copy(0, s).wait()
          else:
            out_copy(0, s).wait()

          if kk == 0:
            idx_fetch(0, ibuf[hh], sem_i[hh]).wait()

          gather(2 * kk, 2 * s, hh).start()
          gather(2 * kk + 1, 2 * s + 1, hh).start()

          sp = 1 - s
          prev_gc0 = w_chunk0 + (blk - 1) * 2
          if hh == 0 and kk == 0:
            @pl.when(g2 > 0)
            def _():
              gather_wait(2 * sp)
              gather_wait(2 * sp + 1)
              out_copy(prev_gc0, sp).start()
          else:
            gather_wait(2 * sp)
            gather_wait(2 * sp + 1)
            out_copy(prev_gc0, sp).start()

          if kk == 0:
            if hh == 0:
              idx_fetch(pair + 1, ibuf[1], sem_i[1]).start()
            else:
              @pl.when(g2 < n_g2 - 1)
              def _():
                idx_fetch(pair + 1, ibuf[0], sem_i[0]).start()
      return 0

    lax.fori_loop(0, n_g2, body, 0)

    last_gc0 = w_chunk0 + c_per_w - 2
    gather_wait(2)
    gather_wait(3)
    out_copy(last_gc0, 1).start()
    out_copy(0, 0).wait()
    out_copy(0, 1).wait()

  return k


def kernel(x, table):
  batch, seqlen = x.shape
  n_total = batch * seqlen
  idx = x.reshape((n_total // _CHUNK, _CHUNK))
  out = _sc_gather(n_total)(idx, table)
  return out.reshape((batch, seqlen, EMBED_DIM))


# speedup vs baseline: 19.6278x; 1.0012x over previous
import functools

import jax
import jax.numpy as jnp
from jax import lax
from jax.experimental import pallas as pl
from jax.experimental.pallas import tpu as pltpu
from jax.experimental.pallas import tpu_sc as plsc

NUM_CODONS = 64
EMBED_DIM = 128

_INFO = plsc.get_sparse_core_info()
_NC = _INFO.num_cores
_NS = _INFO.num_subcores
_NW = _NC * _NS

_CHUNK = 128


def _sc_gather(n_total: int):
  n_chunks = n_total // _CHUNK
  c_per_w = n_chunks // _NW
  n_pairs = c_per_w // 8
  n_g2 = n_pairs // 2
  mesh = plsc.VectorSubcoreMesh(core_axis_name="c", subcore_axis_name="s")

  @functools.partial(
      pl.kernel,
      mesh=mesh,
      out_type=jax.ShapeDtypeStruct((n_chunks, _CHUNK, EMBED_DIM), jnp.float32),
      scratch_types=(
          [pltpu.VMEM((4, _CHUNK, EMBED_DIM), jnp.float32)]
          + [pltpu.VMEM((8, _CHUNK), jnp.int32)] * 2
          + [pltpu.VMEM_SHARED((4, NUM_CODONS, EMBED_DIM), jnp.float32)]
          + [pltpu.SemaphoreType.DMA] * (4 + 2 + 2)
      ),
  )
  def k(idx_hbm, table_hbm, out_hbm, rows, ibuf0, ibuf1, table_sh, *sems):
    ibuf = (ibuf0, ibuf1)
    sem_g = sems[:4]
    sem_o = sems[4:6]
    sem_i = sems[6:]

    sid = lax.axis_index("s")
    wid = sid * _NC + lax.axis_index("c")
    w_chunk0 = wid * c_per_w
    my_tab = table_sh.at[sid % 4]

    def idx_fetch(pair, buf, sem):
      return pltpu.make_async_copy(
          idx_hbm.at[pl.ds(w_chunk0 + pair * 8, 8)], buf, sem)

    def gather(j, slot, hh):
      return pltpu.make_async_copy(
          my_tab.at[ibuf[hh].at[j]], rows.at[slot], sem_g[slot])

    def gather_wait(slot):
      pltpu.make_async_copy(
          my_tab.at[ibuf[0].at[0]], rows.at[slot], sem_g[slot]).wait()

    def out_copy(blk_gc0, s):
      return pltpu.make_async_copy(
          rows.at[pl.ds(2 * s, 2)], out_hbm.at[pl.ds(blk_gc0, 2)], sem_o[s])

    @pl.when(sid < 4)
    def _():
      pltpu.sync_copy(table_hbm, table_sh.at[sid % 4])
    plsc.subcore_barrier()

    idx_fetch(0, ibuf[0], sem_i[0]).start()

    def body(g2, _):
      for hh in range(2):
        pair = 2 * g2 + hh
        for kk in range(4):
          s = kk % 2
          blk = pair * 4 + kk
          first_ever = hh == 0 and kk < 2

          if first_ever:
            @pl.when(g2 > 0)
            def _():
              out_copy(0, s).wait()
          else:
            out_copy(0, s).wait()

          if kk == 0:
            idx_fetch(0, ibuf[hh], sem_i[hh]).wait()

          gather(2 * kk, 2 * s, hh).start()
          gather(2 * kk + 1, 2 * s + 1, hh).start()

          sp = 1 - s
          prev_gc0 = w_chunk0 + (blk - 1) * 2
          if hh == 0 and kk == 0:
            @pl.when(g2 > 0)
            def _():
              gather_wait(2 * sp)
              gather_wait(2 * sp + 1)
              out_copy(prev_gc0, sp).start()
          else:
            gather_wait(2 * sp)
            gather_wait(2 * sp + 1)
            out_copy(prev_gc0, sp).start()

          if kk == 0:
            if hh == 0:
              idx_fetch(pair + 1, ibuf[1], sem_i[1]).start()
            else:
              @pl.when(g2 < n_g2 - 1)
              def _():
                idx_fetch(pair + 1, ibuf[0], sem_i[0]).start()
      return 0

    lax.fori_loop(0, n_g2, body, 0)

    last_gc0 = w_chunk0 + c_per_w - 2
    gather_wait(2)
    gather_wait(3)
    out_copy(last_gc0, 1).start()
    out_copy(0, 0).wait()
    out_copy(0, 1).wait()

  return k


def kernel(x, table):
  batch, seqlen = x.shape
  n_total = batch * seqlen
  idx = x.reshape((n_total // _CHUNK, _CHUNK))
  out = _sc_gather(n_total)(idx, table)
  return out.reshape((batch, seqlen, EMBED_DIM))
